# Initial kernel scaffold; baseline (speedup 1.0000x reference)
import jax
import jax.numpy as jnp
from jax.experimental import pallas as pl


def kernel(inputs, tok_table, pos_table):
    raise NotImplementedError("write your pallas kernel here")


# speedup vs baseline: 1.1683x; 1.1683x over previous
import functools
import math

import jax
import jax.numpy as jnp
from jax import lax
from jax.experimental import pallas as pl
from jax.experimental.pallas import tpu as pltpu
from jax.experimental.pallas import tpu_sc as plsc

VOCAB = 100000
SEQ_LEN = 2048
D_MODEL = 768
BATCH = 4

NUM_WORKERS = 32
POS_PER_W = SEQ_LEN // NUM_WORKERS
LANES = 16
GROUPS = D_MODEL // LANES
SCALE = math.sqrt(float(D_MODEL))


def _body(idx_hbm, tok_hbm, pos_hbm, out_hbm, idx_v, tok_v, pos_v, sem):
    wid = lax.axis_index("s") * 2 + lax.axis_index("c")
    pos_base = wid * POS_PER_W

    pltpu.sync_copy(pos_hbm.at[pl.ds(pos_base, POS_PER_W)], pos_v)

    for b in range(BATCH):
        off = b * SEQ_LEN + pos_base
        pltpu.sync_copy(idx_hbm.at[pl.ds(off, POS_PER_W)], idx_v)
        pltpu.async_copy(tok_hbm.at[idx_v], tok_v, sem).wait()

        def row(r, _):
            for j in range(GROUPS):
                sl = pl.ds(j * LANES, LANES)
                tok_v[r, sl] = tok_v[r, sl] * SCALE + pos_v[r, sl]
            return _

        lax.fori_loop(0, POS_PER_W, row, 0)
        pltpu.sync_copy(tok_v, out_hbm.at[pl.ds(off, POS_PER_W)])


@jax.jit
def _embed(idx_flat, tok_table, pos_table):
    mesh = plsc.VectorSubcoreMesh(core_axis_name="c", subcore_axis_name="s")
    k = functools.partial(
        pl.kernel,
        mesh=mesh,
        out_type=jax.ShapeDtypeStruct((BATCH * SEQ_LEN, D_MODEL), jnp.float32),
        scratch_types=[
            pltpu.VMEM((POS_PER_W,), jnp.int32),
            pltpu.VMEM((POS_PER_W, D_MODEL), jnp.float32),
            pltpu.VMEM((POS_PER_W, D_MODEL), jnp.float32),
            pltpu.SemaphoreType.DMA,
        ],
    )(_body)
    return k(idx_flat, tok_table, pos_table)


def kernel(inputs, tok_table, pos_table):
    idx_flat = inputs.astype(jnp.int32).reshape(-1)
    out = _embed(idx_flat, tok_table, pos_table)
    return out.reshape(BATCH, SEQ_LEN, D_MODEL)
